# baseline (speedup 1.0000x reference)
import jax
import jax.numpy as jnp
from jax.experimental import pallas as pl
from jax.experimental.pallas import tpu as pltpu

N_EMB = 1024
DIM = 64
COMMITMENT_COST = 0.25
BLK = 8192


def _vq_block(x_ref, c_ref, q_ref, idx_ref, loss_ref, perp_ref,
              counts_ref, cs_ref, n2c_ref):
    i = pl.program_id(0)
    nsteps = pl.num_programs(0)
    total_tokens = nsteps * BLK

    @pl.when(i == 0)
    def _prep():
        c0 = c_ref[...]
        cs_ref[...] = jnp.sum(c0 * c0, axis=1, keepdims=True)
        n2c_ref[...] = c0 * (-2.0)

    x = x_ref[...]

    input_sq = jnp.sum(x * x, axis=1, keepdims=True)
    is_row = input_sq.T
    mm2 = jax.lax.dot_general(n2c_ref[...], x, (((1,), (1,)), ((), ())),
                              preferred_element_type=jnp.float32)
    d = (is_row + cs_ref[...]) + mm2

    dmin = jnp.min(d, axis=0, keepdims=True)
    row = jax.lax.broadcasted_iota(jnp.int32, d.shape, 0)
    idx = jnp.min(jnp.where(d == dmin, row, N_EMB), axis=0)

    oh = (row == idx[None, :]).astype(jnp.float32)
    q = jax.lax.dot_general(oh, c_ref[...], (((0,), (0,)), ((), ())),
                            preferred_element_type=jnp.float32)

    q_ref[...] = q
    idx_ref[0, 0, :] = idx

    blk_loss = jnp.sum(dmin)
    ones_col = jnp.ones((BLK, 1), jnp.float32)
    blk_counts = jnp.dot(oh, ones_col,
                         preferred_element_type=jnp.float32)

    @pl.when(i == 0)
    def _init():
        counts_ref[...] = blk_counts
        loss_ref[...] = blk_loss.reshape(1, 1)
        perp_ref[...] = jnp.zeros((1, 1), jnp.float32)

    @pl.when(i > 0)
    def _acc():
        counts_ref[...] += blk_counts
        loss_ref[...] += blk_loss.reshape(1, 1)

    @pl.when(i == nsteps - 1)
    def _finish():
        p = counts_ref[:, 0] / jnp.float32(total_tokens)
        ent = jnp.sum(p * jnp.log(p + 1e-10))
        perp_ref[...] = jnp.exp(-ent).reshape(1, 1)
        mse = loss_ref[0, 0] / jnp.float32(total_tokens * DIM)
        loss_ref[...] = (mse * COMMITMENT_COST + mse).reshape(1, 1)


def kernel(inputs, codebook):
    input_shape = inputs.shape
    x = inputs.reshape(-1, DIM)
    tokens = x.shape[0]
    grid = tokens // BLK

    q, idx3, vq_loss, perp = pl.pallas_call(
        _vq_block,
        grid=(grid,),
        in_specs=[
            pl.BlockSpec((BLK, DIM), lambda i: (i, 0)),
            pl.BlockSpec((N_EMB, DIM), lambda i: (0, 0)),
        ],
        out_specs=[
            pl.BlockSpec((BLK, DIM), lambda i: (i, 0)),
            pl.BlockSpec((1, 1, BLK), lambda i: (i, 0, 0)),
            pl.BlockSpec((1, 1), lambda i: (0, 0)),
            pl.BlockSpec((1, 1), lambda i: (0, 0)),
        ],
        out_shape=[
            jax.ShapeDtypeStruct((tokens, DIM), jnp.float32),
            jax.ShapeDtypeStruct((grid, 1, BLK), jnp.int32),
            jax.ShapeDtypeStruct((1, 1), jnp.float32),
            jax.ShapeDtypeStruct((1, 1), jnp.float32),
        ],
        scratch_shapes=[
            pltpu.VMEM((N_EMB, 1), jnp.float32),
            pltpu.VMEM((N_EMB, 1), jnp.float32),
            pltpu.VMEM((N_EMB, DIM), jnp.float32),
        ],
    )(x, codebook)

    quantized = q.reshape(input_shape)
    indices = idx3.reshape(input_shape[:-1])
    return (quantized, vq_loss[0, 0], perp[0, 0], indices)
